# baseline (speedup 1.0000x reference)
import jax
import jax.numpy as jnp
from jax.experimental import pallas as pl


def _copy_body(x_ref, o_ref):
    o_ref[...] = x_ref[...]


def kernel(x, dim, index, src):
    B, D = index.shape
    cols = jnp.broadcast_to(jnp.arange(D, dtype=index.dtype)[None, :], (B, D))
    y = x.at[index, cols].add(src)
    R = 20000
    grid = (x.shape[0] // R,)
    return pl.pallas_call(
        _copy_body,
        grid=grid,
        in_specs=[pl.BlockSpec((R, x.shape[1]), lambda i: (i, 0))],
        out_specs=pl.BlockSpec((R, x.shape[1]), lambda i: (i, 0)),
        out_shape=jax.ShapeDtypeStruct(x.shape, x.dtype),
    )(y)


# speedup vs baseline: 7.0500x; 7.0500x over previous
import functools

import jax
import jax.numpy as jnp
from jax import lax
from jax.experimental import pallas as pl
from jax.experimental.pallas import tpu as pltpu
from jax.experimental.pallas import tpu_sc as plsc

NROWS = 1_000_000
D = 64
NELEM = 16384 * 64
NT = 32
EPT = NELEM // NT
W = 1000
NW = NROWS // W
WSZ = W * D
NBINS = 1024
CH = 8192
REGION = 36864
REGION2 = REGION * 2
PAIRS_WORDS = NT * REGION2 + 1024
TBL_WORDS = NT * NBINS
ACS = 64
ROUNDS_FULL = 31
_MESH = dict(core_axis_name="c", subcore_axis_name="s")


def _wid():
    return lax.axis_index("s") * 2 + lax.axis_index("c")


def _scan_base():
    cnt0, _ = plsc.scan_count(jnp.zeros((16,), jnp.int32))
    return jnp.min(cnt0)


def _partition_body(idx_hbm, src_hbm, pairs_out, tbl_out,
                    idx_chunk, src_chunk, hist, cursor, packbuf, part_buf):
    wid = _wid()
    base = wid * EPT
    iota16 = lax.iota(jnp.int32, 16)
    cbase = _scan_base()

    zero16 = jnp.zeros((16,), jnp.int32)
    for i in range(NBINS // 16):
        hist[pl.ds(i * 16, 16)] = zero16

    def hist_chunk(c, _):
        start = pl.multiple_of(base + c * CH, CH)
        pltpu.sync_copy(idx_hbm.at[pl.ds(start, CH)], idx_chunk)

        def hist_vec(v, _):
            rows = idx_chunk[pl.ds(v * 16, 16)]
            b = rows // W
            cnt, last = plsc.scan_count(b)
            plsc.addupdate_scatter(hist, [b], cnt - cbase + 1, mask=last)
            return 0

        return lax.fori_loop(0, CH // 16, hist_vec, 0)

    lax.fori_loop(0, EPT // CH, hist_chunk, 0)

    def prefix(i, carry):
        h = hist[pl.ds(i * 16, 16)]
        rs = (h + 3) & jnp.int32(-4)
        cs = plsc.cumsum(rs)
        offs = cs - rs + carry
        cursor[pl.ds(i * 16, 16)] = offs
        packbuf[pl.ds(i * 16, 16)] = (offs << 16) | h
        return carry + jnp.sum(rs)

    lax.fori_loop(0, NBINS // 16, prefix, jnp.int32(0))
    pltpu.sync_copy(packbuf, tbl_out.at[pl.ds(pl.multiple_of(wid * NBINS, NBINS), NBINS)])

    def place_chunk(c, _):
        start = pl.multiple_of(base + c * CH, CH)
        pltpu.sync_copy(idx_hbm.at[pl.ds(start, CH)], idx_chunk)
        pltpu.sync_copy(src_hbm.at[pl.ds(start, CH)], src_chunk)

        def place_vec(v, _):
            rows = idx_chunk[pl.ds(v * 16, 16)]
            vals = src_chunk[pl.ds(v * 16, 16)]
            b = rows // W
            cnt, last = plsc.scan_count(b)
            rank = cnt - cbase
            dest = plsc.load_gather(cursor, [b]) + rank
            j = (iota16 + v * 16) & 63
            pos = (rows - b * W) * D + j
            plsc.store_scatter(part_buf, [dest * 2], pos)
            plsc.store_scatter(part_buf, [dest * 2 + 1],
                               plsc.bitcast(vals, jnp.int32))
            plsc.store_scatter(cursor, [b], dest + 1, mask=last)
            return 0

        return lax.fori_loop(0, CH // 16, place_vec, 0)

    lax.fori_loop(0, EPT // CH, place_chunk, 0)
    pltpu.sync_copy(part_buf, pairs_out.at[pl.ds(pl.multiple_of(wid * REGION2, 1024), REGION2)])


def _apply_body(x_hbm, pairs_hbm, tbl_hbm, out_hbm,
                xwin, tblidx, tblbuf, chunkbuf, sem):
    wid = _wid()
    iota16 = lax.iota(jnp.int32, 16)
    cbase = _scan_base()

    def do_window(w):
        tblidx[pl.ds(0, 16)] = iota16 * NBINS + w
        tblidx[pl.ds(16, 16)] = (iota16 + 16) * NBINS + w
        pltpu.async_copy(tbl_hbm.at[tblidx], tblbuf.at[pl.ds(0, NT)], sem).wait()
        pltpu.sync_copy(x_hbm.at[pl.ds(pl.multiple_of(w * WSZ, 64), WSZ)], xwin)

        def src_tile(t, _):
            packed = tblbuf[pl.ds(t, 16)][0]
            off = lax.shift_right_logical(packed, 16)
            size = packed & jnp.int32(0xFFFF)
            gbase = t * REGION2 + off * 2

            def chunk_step(done):
                cstart = pl.multiple_of(gbase + done * 2, 8)
                pltpu.sync_copy(
                    pairs_hbm.at[pl.ds(cstart, ACS * 2)], chunkbuf)
                navail = jnp.minimum(size - done, ACS)

                def vec(vi, _):
                    lanes = vi * 16 + iota16
                    valid = lanes < navail
                    pos = plsc.load_gather(chunkbuf, [lanes * 2])
                    val = plsc.bitcast(
                        plsc.load_gather(chunkbuf, [lanes * 2 + 1]),
                        jnp.float32)
                    cnt, _last = plsc.scan_count(pos, mask=valid)
                    rank = jnp.where(valid, cnt - cbase, 0)
                    maxdup = jnp.max(rank)

                    def fast():
                        plsc.addupdate_scatter(xwin, [pos], val, mask=valid)

                    def slow():
                        def dup_round(d):
                            plsc.addupdate_scatter(
                                xwin, [pos], val,
                                mask=valid & (rank == d))
                            return d + 1
                        lax.while_loop(lambda d: d <= maxdup, dup_round,
                                       jnp.int32(0))

                    lax.cond(maxdup > 0, slow, fast)
                    return 0

                nvec = (navail + 15) // 16
                lax.fori_loop(0, nvec, vec, 0)
                return done + ACS

            lax.while_loop(lambda d: d < size, chunk_step, jnp.int32(0))
            return 0

        lax.fori_loop(0, NT, src_tile, 0)
        pltpu.sync_copy(xwin, out_hbm.at[pl.ds(pl.multiple_of(w * WSZ, 64), WSZ)])

    def round_body(r, _):
        do_window(r * NT + wid)
        return 0

    lax.fori_loop(0, ROUNDS_FULL, round_body, 0)

    @pl.when(ROUNDS_FULL * NT + wid < NW)
    def _():
        do_window(ROUNDS_FULL * NT + wid)


@functools.partial(
    pl.kernel,
    out_type=[
        jax.ShapeDtypeStruct((PAIRS_WORDS,), jnp.int32),
        jax.ShapeDtypeStruct((TBL_WORDS,), jnp.int32),
    ],
    mesh=plsc.VectorSubcoreMesh(**_MESH),
    compiler_params=pltpu.CompilerParams(needs_layout_passes=False),
    scratch_types=[
        pltpu.VMEM((CH,), jnp.int32),
        pltpu.VMEM((CH,), jnp.float32),
        pltpu.VMEM((NBINS,), jnp.int32),
        pltpu.VMEM((NBINS,), jnp.int32),
        pltpu.VMEM((NBINS,), jnp.int32),
        pltpu.VMEM((REGION2,), jnp.int32),
    ],
)
def _partition(idx_hbm, src_hbm, pairs_out, tbl_out, *scratch):
    _partition_body(idx_hbm, src_hbm, pairs_out, tbl_out, *scratch)


@functools.partial(
    pl.kernel,
    out_type=jax.ShapeDtypeStruct((NROWS * D,), jnp.float32),
    mesh=plsc.VectorSubcoreMesh(**_MESH),
    compiler_params=pltpu.CompilerParams(needs_layout_passes=False),
    scratch_types=[
        pltpu.VMEM((WSZ,), jnp.float32),
        pltpu.VMEM((NT,), jnp.int32),
        pltpu.VMEM((NT + 16,), jnp.int32),
        pltpu.VMEM((ACS * 2,), jnp.int32),
        pltpu.SemaphoreType.DMA,
    ],
)
def _apply(x_hbm, pairs_hbm, tbl_hbm, out_hbm, *scratch):
    _apply_body(x_hbm, pairs_hbm, tbl_hbm, out_hbm, *scratch)


def kernel(x, dim, index, src):
    del dim
    idx_flat = index.reshape(-1).astype(jnp.int32)
    src_flat = src.reshape(-1)
    pairs, tbl = _partition(idx_flat, src_flat)
    out_flat = _apply(x.reshape(-1), pairs, tbl)
    return out_flat.reshape(x.shape)


# speedup vs baseline: 10.8632x; 1.5409x over previous
import functools

import jax
import jax.numpy as jnp
from jax import lax
from jax.experimental import pallas as pl
from jax.experimental.pallas import tpu as pltpu
from jax.experimental.pallas import tpu_sc as plsc

NROWS = 1_000_000
D = 64
NELEM = 16384 * 64
NT = 32
EPT = NELEM // NT
W = 512
NW = 1954
WLAST = 64
NBINS = 2048
CH = 8192
REGION = 40960
REGION2 = REGION * 2
PAIRS_WORDS = NT * REGION2 + 1024
TBL_WORDS = NT * NBINS
ACS = 64
ROUNDS_FULL = 61
_MESH = dict(core_axis_name="c", subcore_axis_name="s")


def _wid():
    return lax.axis_index("s") * 2 + lax.axis_index("c")


def _scan_base():
    cnt0, _ = plsc.scan_count(jnp.zeros((16,), jnp.int32))
    return jnp.min(cnt0)


def _partition_body(idx_hbm, src_hbm, pairs_out, tbl_out,
                    idx_chunk, src_chunk, hist, cursor, packbuf, part_buf):
    wid = _wid()
    base = wid * EPT
    iota16 = lax.iota(jnp.int32, 16)
    cbase = _scan_base()

    zero16 = jnp.zeros((16,), jnp.int32)
    for i in range(NBINS // 16):
        hist[pl.ds(i * 16, 16)] = zero16

    def hist_chunk(c, _):
        start = pl.multiple_of(base + c * CH, CH)
        pltpu.sync_copy(idx_hbm.at[pl.ds(start, CH)], idx_chunk)

        def hist_vec(v, _):
            rows = idx_chunk[pl.ds(v * 16, 16)]
            b = lax.shift_right_logical(rows, 9)
            cnt, last = plsc.scan_count(b)
            plsc.addupdate_scatter(hist, [b], cnt - cbase + 1, mask=last)
            return 0

        return lax.fori_loop(0, CH // 16, hist_vec, 0)

    lax.fori_loop(0, EPT // CH, hist_chunk, 0)

    def prefix(i, carry):
        h = hist[pl.ds(i * 16, 16)]
        rs = (h + 3) & jnp.int32(-4)
        cs = plsc.cumsum(rs)
        offs = cs - rs + carry
        cursor[pl.ds(i * 16, 16)] = offs
        packbuf[pl.ds(i * 16, 16)] = (offs << 16) | h
        return carry + jnp.sum(rs)

    lax.fori_loop(0, NBINS // 16, prefix, jnp.int32(0))
    pltpu.sync_copy(packbuf, tbl_out.at[pl.ds(pl.multiple_of(wid * NBINS, NBINS), NBINS)])

    def place_chunk(c, _):
        start = pl.multiple_of(base + c * CH, CH)
        pltpu.sync_copy(idx_hbm.at[pl.ds(start, CH)], idx_chunk)
        pltpu.sync_copy(src_hbm.at[pl.ds(start, CH)], src_chunk)

        def place_vec(v, _):
            rows = idx_chunk[pl.ds(v * 16, 16)]
            vals = src_chunk[pl.ds(v * 16, 16)]
            b = lax.shift_right_logical(rows, 9)
            cnt, last = plsc.scan_count(b)
            rank = cnt - cbase
            dest = plsc.load_gather(cursor, [b]) + rank
            j = (iota16 + v * 16) & 63
            pos = ((rows & jnp.int32(511)) << 6) | j
            plsc.store_scatter(part_buf, [dest * 2], pos)
            plsc.store_scatter(part_buf, [dest * 2 + 1],
                               plsc.bitcast(vals, jnp.int32))
            plsc.store_scatter(cursor, [b], dest + 1, mask=last)
            return 0

        return lax.fori_loop(0, CH // 16, place_vec, 0)

    lax.fori_loop(0, EPT // CH, place_chunk, 0)
    pltpu.sync_copy(part_buf, pairs_out.at[pl.ds(pl.multiple_of(wid * REGION2, 1024), REGION2)])


def _apply_body(x_hbm, pairs_hbm, tbl_hbm, out_hbm,
                xwin, tblidx, tblbuf, segbuf, chunkbuf, sem, sem2):
    wid = _wid()
    iota16 = lax.iota(jnp.int32, 16)
    cbase = _scan_base()

    def apply_vectors(buf, boff, navail):
        def vec(vi, _):
            lanes = vi * 16 + iota16
            valid = lanes < navail
            pos = plsc.load_gather(buf, [boff + lanes * 2])
            val = plsc.bitcast(
                plsc.load_gather(buf, [boff + lanes * 2 + 1]), jnp.float32)
            cnt, _last = plsc.scan_count(pos, mask=valid)
            rank = jnp.where(valid, cnt - cbase, 0)
            maxdup = jnp.max(rank)
            r = lax.shift_right_logical(pos, 6)
            c = pos & jnp.int32(63)

            def fast():
                plsc.addupdate_scatter(xwin, [r, c], val, mask=valid)

            def slow():
                def dup_round(d):
                    plsc.addupdate_scatter(
                        xwin, [r, c], val, mask=valid & (rank == d))
                    return d + 1
                lax.while_loop(lambda d: d <= maxdup, dup_round, jnp.int32(0))

            lax.cond(maxdup > 0, slow, fast)
            return 0

        nvec = (navail + 15) // 16
        lax.fori_loop(0, nvec, vec, 0)

    def do_window(w, nrows):
        tblidx[pl.ds(0, 16)] = iota16 * NBINS + w
        tblidx[pl.ds(16, 16)] = (iota16 + 16) * NBINS + w
        pltpu.async_copy(tbl_hbm.at[tblidx], tblbuf.at[pl.ds(0, NT)], sem).wait()
        rstart = pl.multiple_of(w * W, W)
        xcopy = pltpu.async_copy(x_hbm.at[pl.ds(rstart, nrows)],
                                 xwin.at[pl.ds(0, nrows)], sem)

        def fire(t, _):
            packed = tblbuf[pl.ds(t, 16)][0]
            off = lax.shift_right_logical(packed, 16)
            gbase = pl.multiple_of(t * REGION2 + off * 2, 8)
            pltpu.async_copy(pairs_hbm.at[pl.ds(gbase, ACS * 2)],
                             segbuf.at[pl.ds(t * ACS * 2, ACS * 2)], sem2)
            return 0

        lax.fori_loop(0, NT, fire, 0)
        pltpu.make_async_copy(pairs_hbm.at[pl.ds(0, NT * ACS * 2)],
                              segbuf, sem2).wait()
        xcopy.wait()

        def src_tile(t, _):
            packed = tblbuf[pl.ds(t, 16)][0]
            off = lax.shift_right_logical(packed, 16)
            size = packed & jnp.int32(0xFFFF)
            apply_vectors(segbuf, t * ACS * 2, jnp.minimum(size, ACS))
            gbase = t * REGION2 + off * 2

            def chunk_step(done):
                cstart = pl.multiple_of(gbase + done * 2, 8)
                pltpu.sync_copy(pairs_hbm.at[pl.ds(cstart, ACS * 2)], chunkbuf)
                apply_vectors(chunkbuf, 0, jnp.minimum(size - done, ACS))
                return done + ACS

            lax.while_loop(lambda d: d < size, chunk_step, jnp.int32(ACS))
            return 0

        lax.fori_loop(0, NT, src_tile, 0)
        pltpu.sync_copy(xwin.at[pl.ds(0, nrows)],
                        out_hbm.at[pl.ds(rstart, nrows)])

    def round_body(r, _):
        do_window(r * NT + wid, W)
        return 0

    lax.fori_loop(0, ROUNDS_FULL, round_body, 0)

    @pl.when(wid == 0)
    def _():
        do_window(jnp.int32(ROUNDS_FULL * NT), W)

    @pl.when(wid == 1)
    def _():
        do_window(jnp.int32(ROUNDS_FULL * NT + 1), WLAST)


@functools.partial(
    pl.kernel,
    out_type=[
        jax.ShapeDtypeStruct((PAIRS_WORDS,), jnp.int32),
        jax.ShapeDtypeStruct((TBL_WORDS,), jnp.int32),
    ],
    mesh=plsc.VectorSubcoreMesh(**_MESH),
    compiler_params=pltpu.CompilerParams(needs_layout_passes=False),
    scratch_types=[
        pltpu.VMEM((CH,), jnp.int32),
        pltpu.VMEM((CH,), jnp.float32),
        pltpu.VMEM((NBINS,), jnp.int32),
        pltpu.VMEM((NBINS,), jnp.int32),
        pltpu.VMEM((NBINS,), jnp.int32),
        pltpu.VMEM((REGION2,), jnp.int32),
    ],
)
def _partition(idx_hbm, src_hbm, pairs_out, tbl_out, *scratch):
    _partition_body(idx_hbm, src_hbm, pairs_out, tbl_out, *scratch)


@functools.partial(
    pl.kernel,
    out_type=jax.ShapeDtypeStruct((NROWS, D), jnp.float32),
    mesh=plsc.VectorSubcoreMesh(**_MESH),
    compiler_params=pltpu.CompilerParams(needs_layout_passes=False),
    scratch_types=[
        pltpu.VMEM((W, D), jnp.float32),
        pltpu.VMEM((NT,), jnp.int32),
        pltpu.VMEM((NT + 16,), jnp.int32),
        pltpu.VMEM((NT * ACS * 2,), jnp.int32),
        pltpu.VMEM((ACS * 2,), jnp.int32),
        pltpu.SemaphoreType.DMA,
        pltpu.SemaphoreType.DMA,
    ],
)
def _apply(x_hbm, pairs_hbm, tbl_hbm, out_hbm, *scratch):
    _apply_body(x_hbm, pairs_hbm, tbl_hbm, out_hbm, *scratch)


def kernel(x, dim, index, src):
    del dim
    idx_flat = index.reshape(-1).astype(jnp.int32)
    src_flat = src.reshape(-1)
    pairs, tbl = _partition(idx_flat, src_flat)
    return _apply(x, pairs, tbl)
